# baseline (speedup 1.0000x reference)
import functools

import jax
import jax.numpy as jnp
from jax import lax
from jax.experimental import pallas as pl
from jax.experimental.pallas import tpu as pltpu
from jax.experimental.pallas import tpu_sc as plsc

_B = 1024
_CTX = 20
_D = 128
_V = 100000

_NW = 32
_BPW = _B // _NW
_RPW = _BPW * _CTX
_ICH = _RPW // 128
_LANES = 16


@functools.lru_cache(maxsize=None)
def _build_gather_sum():
    mesh = plsc.VectorSubcoreMesh(core_axis_name="c", subcore_axis_name="s")
    return functools.partial(
        pl.kernel,
        mesh=mesh,
        out_type=jax.ShapeDtypeStruct((_B, _D), jnp.float32),
        scratch_types=[
            pltpu.VMEM((_RPW,), jnp.int32),
            pltpu.VMEM((_RPW, _D), jnp.float32),
            pltpu.VMEM((_BPW, _D), jnp.float32),
            pltpu.SemaphoreType.DMA,
        ],
    )(_gather_sum_body)


def _gather_sum_body(idx_hbm, table_hbm, out_hbm, idx_v, rows_v, acc_v, sem):
    wid = lax.axis_index("s") * 2 + lax.axis_index("c")
    pltpu.sync_copy(idx_hbm.at[pl.ds(wid * _RPW, _RPW)], idx_v)
    copies = [
        pltpu.async_copy(
            table_hbm.at[idx_v.at[pl.ds(j * 128, 128)]],
            rows_v.at[pl.ds(j * 128, 128)],
            sem,
        )
        for j in range(_ICH)
    ]
    for cp in copies:
        cp.wait()

    def body(r, carry):
        base = r * _CTX
        for c in range(_D // _LANES):
            acc = rows_v[base, pl.ds(c * _LANES, _LANES)]
            for j in range(1, _CTX):
                acc = acc + rows_v[base + j, pl.ds(c * _LANES, _LANES)]
            acc_v[r, pl.ds(c * _LANES, _LANES)] = acc
        return carry

    lax.fori_loop(0, _BPW, body, 0)
    pltpu.sync_copy(acc_v, out_hbm.at[pl.ds(wid * _BPW, _BPW)])


_BV = 512


def _proj_body(x_ref, w_ref, b_ref, o_ref):
    o_ref[...] = (
        jnp.dot(x_ref[...], w_ref[...], preferred_element_type=jnp.float32)
        + b_ref[...]
    )


def _project(ctx_sum, W, b2):
    nblk = pl.cdiv(_V, _BV)
    return pl.pallas_call(
        _proj_body,
        grid=(nblk,),
        in_specs=[
            pl.BlockSpec((_B, _D), lambda i: (0, 0)),
            pl.BlockSpec((_D, _BV), lambda i: (0, i)),
            pl.BlockSpec((1, _BV), lambda i: (0, i)),
        ],
        out_specs=pl.BlockSpec((_B, _BV), lambda i: (0, i)),
        out_shape=jax.ShapeDtypeStruct((_B, _V), jnp.float32),
    )(ctx_sum, W, b2)


def kernel(context_words, emb_table, W, b):
    idx = context_words.astype(jnp.int32).reshape(_B * _CTX)
    ctx_sum = _build_gather_sum()(idx, emb_table)
    return _project(ctx_sum, W, b.reshape(1, _V))


# speedup vs baseline: 1.1450x; 1.1450x over previous
import functools

import jax
import jax.numpy as jnp
from jax import lax
from jax.experimental import pallas as pl
from jax.experimental.pallas import tpu as pltpu
from jax.experimental.pallas import tpu_sc as plsc

_B = 1024
_CTX = 20
_D = 128
_V = 100000

_NW = 32
_BPW = _B // _NW
_RPW = _BPW * _CTX
_ICH = _RPW // 128
_LANES = 16


@functools.lru_cache(maxsize=None)
def _build_gather_sum():
    mesh = plsc.VectorSubcoreMesh(core_axis_name="c", subcore_axis_name="s")
    return functools.partial(
        pl.kernel,
        mesh=mesh,
        out_type=jax.ShapeDtypeStruct((_B, _D), jnp.float32),
        scratch_types=[
            pltpu.VMEM((_RPW,), jnp.int32),
            pltpu.VMEM((_RPW, _D), jnp.float32),
            pltpu.VMEM((_BPW, _D), jnp.float32),
            pltpu.SemaphoreType.DMA,
        ],
    )(_gather_sum_body)


def _gather_sum_body(idx_hbm, table_hbm, out_hbm, idx_v, rows_v, acc_v, sem):
    wid = lax.axis_index("s") * 2 + lax.axis_index("c")
    pltpu.sync_copy(idx_hbm.at[pl.ds(wid * _RPW, _RPW)], idx_v)
    copies = [
        pltpu.async_copy(
            table_hbm.at[idx_v.at[pl.ds(j * 128, 128)]],
            rows_v.at[pl.ds(j * 128, 128)],
            sem,
        )
        for j in range(_ICH)
    ]
    for cp in copies:
        cp.wait()

    def body(r, carry):
        base = r * _CTX
        for c in range(_D // _LANES):
            acc = rows_v[base, pl.ds(c * _LANES, _LANES)]
            for j in range(1, _CTX):
                acc = acc + rows_v[base + j, pl.ds(c * _LANES, _LANES)]
            acc_v[r, pl.ds(c * _LANES, _LANES)] = acc
        return carry

    lax.fori_loop(0, _BPW, body, 0)
    pltpu.sync_copy(acc_v, out_hbm.at[pl.ds(wid * _BPW, _BPW)])


_BV = 2048
_NBLK = pl.cdiv(_V, _BV)
_NBUF = 4
_VPAD = _NBLK * _BV


def _proj_body(x_ref, w_ref, b_ref, o_ref):
    x_bf = x_ref[...].astype(jnp.bfloat16)
    w_bf = w_ref[...].astype(jnp.bfloat16)
    o_ref[...] = (
        jnp.dot(x_bf, w_bf, preferred_element_type=jnp.float32)
        + b_ref[...]
    )


def _project(ctx_sum, W, b2):
    return pl.pallas_call(
        _proj_body,
        grid=(_NBLK,),
        in_specs=[
            pl.BlockSpec((_B, _D), lambda i: (0, 0)),
            pl.BlockSpec((_D, _BV), lambda i: (0, i)),
            pl.BlockSpec((1, _BV), lambda i: (0, i)),
        ],
        out_specs=pl.BlockSpec((_B, _BV), lambda i: (0, i)),
        out_shape=jax.ShapeDtypeStruct((_B, _V), jnp.float32),
    )(ctx_sum, W, b2)


def kernel(context_words, emb_table, W, b):
    idx = context_words.astype(jnp.int32).reshape(_B * _CTX)
    ctx_sum = _build_gather_sum()(idx, emb_table)
    return _project(ctx_sum, W, b.reshape(1, _V))
